# baseline (speedup 1.0000x reference)
import jax
import jax.numpy as jnp
from jax import lax
from jax.experimental import pallas as pl
from jax.experimental.pallas import tpu as pltpu
from jax.experimental.pallas import tpu_sc as plsc

N_FACTORS = 32
BATCH = 16384
NUM_CORES = 2
NUM_SUBCORES = 16
NW = NUM_CORES * NUM_SUBCORES
BPW = BATCH // NW
CHUNK = 128
NCH = BPW // CHUNK
LANES = 16


def _pmf_body(uidx_hbm, iidx_hbm, uemb_hbm, iemb_hbm, ub_hbm, ib_hbm,
              out_hbm, uidx_v, iidx_v, urows_v, irows_v, ubv, ibv, outv,
              sem):
    wid = lax.axis_index("s") * NUM_CORES + lax.axis_index("c")

    pltpu.sync_copy(uidx_hbm.at[wid], uidx_v)
    pltpu.sync_copy(iidx_hbm.at[wid], iidx_v)

    copies = []
    for j in range(NCH):
        sl = pl.ds(j * CHUNK, CHUNK)
        copies.append(pltpu.async_copy(uemb_hbm.at[uidx_v.at[j]],
                                       urows_v.at[sl], sem))
        copies.append(pltpu.async_copy(iemb_hbm.at[iidx_v.at[j]],
                                       irows_v.at[sl], sem))
        copies.append(pltpu.async_copy(ub_hbm.at[uidx_v.at[j]],
                                       ubv.at[sl], sem))
        copies.append(pltpu.async_copy(ib_hbm.at[iidx_v.at[j]],
                                       ibv.at[sl], sem))
    for c in copies:
        c.wait()

    def block(blk, carry):
        base = blk * LANES
        rows = base + lax.iota(jnp.int32, LANES)
        acc = ubv[pl.ds(base, LANES)] + ibv[pl.ds(base, LANES)]
        for f in range(N_FACTORS):
            cols = jnp.full((LANES,), f, jnp.int32)
            uv = plsc.load_gather(urows_v, [rows, cols])
            iv = plsc.load_gather(irows_v, [rows, cols])
            acc = acc + uv * iv
        outv[pl.ds(base, LANES)] = acc
        return carry

    lax.fori_loop(0, BPW // LANES, block, 0)

    pltpu.sync_copy(outv, out_hbm.at[pl.ds(wid * BPW, BPW)])


def kernel(users_index, items_index, user_emb, item_emb, ub, ib):
    uidx = users_index.astype(jnp.int32).reshape(NW, NCH, CHUNK)
    iidx = items_index.astype(jnp.int32).reshape(NW, NCH, CHUNK)
    ubf = ub.reshape(-1)
    ibf = ib.reshape(-1)

    mesh = plsc.VectorSubcoreMesh(core_axis_name="c", subcore_axis_name="s")

    run = pl.kernel(
        _pmf_body,
        mesh=mesh,
        out_type=jax.ShapeDtypeStruct((BATCH,), jnp.float32),
        scratch_types=[
            pltpu.VMEM((NCH, CHUNK), jnp.int32),
            pltpu.VMEM((NCH, CHUNK), jnp.int32),
            pltpu.VMEM((BPW, N_FACTORS), jnp.float32),
            pltpu.VMEM((BPW, N_FACTORS), jnp.float32),
            pltpu.VMEM((BPW,), jnp.float32),
            pltpu.VMEM((BPW,), jnp.float32),
            pltpu.VMEM((BPW,), jnp.float32),
            pltpu.SemaphoreType.DMA,
        ],
        compiler_params=pltpu.CompilerParams(
            needs_layout_passes=False, use_tc_tiling_on_sc=False),
    )
    return run(uidx, iidx, user_emb, item_emb, ubf, ibf)
